# baseline (speedup 1.0000x reference)
import functools

import jax
import jax.numpy as jnp
from jax import lax
from jax.experimental import pallas as pl
from jax.experimental.pallas import tpu as pltpu
from jax.experimental.pallas import tpu_sc as plsc

EMBED_DIM = 64
OUT_DIM = 2
NC = 2
NS = 16
NW = NC * NS
PROJ_BLK = 16384


def _project_body(table_ref, w_ref, b_ref, out_ref):
    x = table_ref[...].astype(jnp.bfloat16)
    w = w_ref[...]
    z = lax.dot_general(w, x, (((1,), (1,)), ((), ())),
                        preferred_element_type=jnp.float32)
    out_ref[...] = jnp.tanh(z + b_ref[...])


@functools.cache
def _make_project(n_rows: int):
    grid = (n_rows + PROJ_BLK - 1) // PROJ_BLK
    return pl.pallas_call(
        _project_body,
        grid=(grid,),
        in_specs=[
            pl.BlockSpec((PROJ_BLK, EMBED_DIM), lambda i: (i, 0)),
            pl.BlockSpec((OUT_DIM, EMBED_DIM), lambda i: (0, 0)),
            pl.BlockSpec((OUT_DIM, 1), lambda i: (0, 0)),
        ],
        out_specs=pl.BlockSpec((OUT_DIM, PROJ_BLK), lambda i: (0, i)),
        out_shape=jax.ShapeDtypeStruct((OUT_DIM, n_rows), jnp.float32),
    )


G = 128


@functools.cache
def _make_gather(pw: int):
    ch = pw // G
    mesh = plsc.VectorSubcoreMesh(core_axis_name="c", subcore_axis_name="s")

    @functools.partial(
        pl.kernel,
        out_type=jax.ShapeDtypeStruct((NW, pw, OUT_DIM), jnp.float32),
        mesh=mesh,
        scratch_types=[
            pltpu.VMEM((ch, G), jnp.int32),
            pltpu.VMEM((pw, OUT_DIM), jnp.float32),
            pltpu.SemaphoreType.DMA,
        ],
        compiler_params=pltpu.CompilerParams(use_tc_tiling_on_sc=False),
    )
    def gather2(idx_hbm, p_hbm, out_hbm, idx_v, rows_v, sem):
        wid = lax.axis_index("s") * NC + lax.axis_index("c")
        pltpu.sync_copy(idx_hbm.at[wid], idx_v)

        def fire(j, _):
            pltpu.async_copy(
                p_hbm.at[idx_v.at[j]], rows_v.at[pl.ds(j * G, G)], sem)
            return 0

        lax.fori_loop(0, ch, fire, 0)

        def drain(j, _):
            pltpu.make_async_copy(
                p_hbm.at[idx_v.at[0]], rows_v.at[pl.ds(0, G)], sem).wait()
            return 0

        lax.fori_loop(0, ch, drain, 0)
        pltpu.sync_copy(rows_v, out_hbm.at[wid])

    return gather2


def kernel(data, table, W, b):
    batch, fields = data.shape
    n = batch * fields
    pw = n // NW
    proj_t = _make_project(table.shape[0])(
        table, W.astype(jnp.bfloat16), b.reshape(OUT_DIM, 1))
    proj = proj_t.T
    idx3 = data.astype(jnp.int32).reshape(NW, pw // G, G)
    out = _make_gather(pw)(idx3, proj)
    return out.reshape(batch, fields, OUT_DIM)


# speedup vs baseline: 1.7658x; 1.7658x over previous
import functools

import jax
import jax.numpy as jnp
from jax import lax
from jax.experimental import pallas as pl
from jax.experimental.pallas import tpu as pltpu
from jax.experimental.pallas import tpu_sc as plsc

EMBED_DIM = 64
OUT_DIM = 2
LANES = 16
NC = 2
NS = 16
NW = NC * NS
G = 128


def _tanh16(z):
    a = jnp.abs(z)
    e = jnp.exp(-2.0 * a)
    return jnp.sign(z) * (1.0 - e) / (1.0 + e)


@functools.cache
def _make_fused(pw: int):
    ch = pw // G
    half = ch // 2

    mesh = plsc.VectorSubcoreMesh(core_axis_name="c", subcore_axis_name="s")

    @functools.partial(
        pl.kernel,
        out_type=jax.ShapeDtypeStruct((NW, pw * OUT_DIM), jnp.float32),
        mesh=mesh,
        scratch_types=[
            pltpu.VMEM((ch, G), jnp.int32),
            pltpu.VMEM((G, EMBED_DIM), jnp.float32),
            pltpu.VMEM((G, EMBED_DIM), jnp.float32),
            pltpu.VMEM((pw * OUT_DIM,), jnp.float32),
            pltpu.VMEM((8, LANES), jnp.float32),
            pltpu.VMEM((LANES,), jnp.float32),
            pltpu.SemaphoreType.DMA,
            pltpu.SemaphoreType.DMA,
        ],
        compiler_params=pltpu.CompilerParams(
            use_tc_tiling_on_sc=False, needs_layout_passes=False),
    )
    def fused(idx_hbm, table_hbm, w_hbm, b_hbm, out_hbm,
              idx_v, rows0, rows1, out_v, w_v, b_v, sem0, sem1):
        wid = lax.axis_index("s") * NC + lax.axis_index("c")
        pltpu.sync_copy(idx_hbm.at[wid], idx_v)
        pltpu.sync_copy(w_hbm, w_v)
        pltpu.sync_copy(b_hbm, b_v)
        w00 = w_v[0]
        w01 = w_v[1]
        w02 = w_v[2]
        w03 = w_v[3]
        w10 = w_v[4]
        w11 = w_v[5]
        w12 = w_v[6]
        w13 = w_v[7]

        bv = b_v[...]
        lane = lax.iota(jnp.int32, LANES)
        masks = [lane == k for k in range(LANES)]

        def start(j, buf, sem):
            pltpu.async_copy(table_hbm.at[idx_v.at[j]], buf, sem)

        def wait(j, buf, sem):
            pltpu.make_async_copy(table_hbm.at[idx_v.at[j]], buf, sem).wait()

        def compute(j, buf):
            def gbody(g, _):
                r0 = g * LANES
                z0 = jnp.zeros((LANES,), jnp.float32)
                z1 = jnp.zeros((LANES,), jnp.float32)
                for rl in range(LANES):
                    v0 = buf[r0 + rl, pl.ds(0, LANES)]
                    v1 = buf[r0 + rl, pl.ds(LANES, LANES)]
                    v2 = buf[r0 + rl, pl.ds(2 * LANES, LANES)]
                    v3 = buf[r0 + rl, pl.ds(3 * LANES, LANES)]
                    a0 = v0 * w00 + v1 * w01 + v2 * w02 + v3 * w03
                    a1 = v0 * w10 + v1 * w11 + v2 * w12 + v3 * w13
                    s0 = jnp.sum(a0)
                    s1 = jnp.sum(a1)
                    k = 2 * (rl % 8)
                    if rl < 8:
                        z0 = jnp.where(masks[k], s0, z0)
                        z0 = jnp.where(masks[k + 1], s1, z0)
                    else:
                        z1 = jnp.where(masks[k], s0, z1)
                        z1 = jnp.where(masks[k + 1], s1, z1)
                base = (j * G + r0) * OUT_DIM
                out_v[pl.ds(base, LANES)] = _tanh16(z0 + bv)
                out_v[pl.ds(base + LANES, LANES)] = _tanh16(z1 + bv)
                return 0

            lax.fori_loop(0, G // LANES, gbody, 0)

        start(0, rows0, sem0)

        def chunk2(jj, _):
            j0 = 2 * jj
            wait(j0, rows0, sem0)
            start(j0 + 1, rows1, sem1)
            compute(j0, rows0)
            wait(j0 + 1, rows1, sem1)

            @pl.when(jj + 1 < half)
            def _prefetch():
                start(j0 + 2, rows0, sem0)

            compute(j0 + 1, rows1)
            return 0

        lax.fori_loop(0, half, chunk2, 0)

        pltpu.sync_copy(out_v, out_hbm.at[wid])

    return fused


def kernel(data, table, W, b):
    batch, fields = data.shape
    n = batch * fields
    pw = n // NW
    idx3 = data.astype(jnp.int32).reshape(NW, pw // G, G)
    wv = W.reshape(OUT_DIM * EMBED_DIM // LANES, LANES)
    bv = jnp.tile(b, LANES // OUT_DIM)
    out = _make_fused(pw)(idx3, table, wv, bv)
    return out.reshape(batch, fields, OUT_DIM)


# speedup vs baseline: 10.9297x; 6.1897x over previous
import functools

import jax
import jax.numpy as jnp
from jax import lax
from jax.experimental import pallas as pl
from jax.experimental.pallas import tpu as pltpu
from jax.experimental.pallas import tpu_sc as plsc

EMBED_DIM = 64
OUT_DIM = 2
NC = 2
NS = 16
NW = NC * NS
PROJ_BLK = 32768
G = 128


def _project_body(tt_ref, w_ref, b_ref, p0_ref, p1_ref):
    x = tt_ref[...].astype(jnp.bfloat16)
    z = lax.dot_general(w_ref[...], x, (((1,), (0,)), ((), ())),
                        preferred_element_type=jnp.float32)
    t = jnp.tanh(z + b_ref[...])
    p0_ref[...] = t[0]
    p1_ref[...] = t[1]


@functools.cache
def _make_project(n_rows: int):
    grid = (n_rows + PROJ_BLK - 1) // PROJ_BLK
    return pl.pallas_call(
        _project_body,
        grid=(grid,),
        in_specs=[
            pl.BlockSpec((EMBED_DIM, PROJ_BLK), lambda i: (0, i)),
            pl.BlockSpec((OUT_DIM, EMBED_DIM), lambda i: (0, 0)),
            pl.BlockSpec((OUT_DIM, 1), lambda i: (0, 0)),
        ],
        out_specs=[
            pl.BlockSpec((PROJ_BLK,), lambda i: (i,)),
            pl.BlockSpec((PROJ_BLK,), lambda i: (i,)),
        ],
        out_shape=[
            jax.ShapeDtypeStruct((n_rows,), jnp.float32),
            jax.ShapeDtypeStruct((n_rows,), jnp.float32),
        ],
    )


@functools.cache
def _make_gather(fields: int, batch: int):
    bw = batch // NW
    qn = bw // G
    ch = fields * qn

    mesh = plsc.VectorSubcoreMesh(core_axis_name="c", subcore_axis_name="s")

    @functools.partial(
        pl.kernel,
        out_type=jax.ShapeDtypeStruct((fields, OUT_DIM, batch), jnp.float32),
        mesh=mesh,
        scratch_types=[
            pltpu.VMEM((fields, bw), jnp.int32),
            pltpu.VMEM((fields, OUT_DIM, bw), jnp.float32),
            pltpu.SemaphoreType.DMA,
        ],
        compiler_params=pltpu.CompilerParams(
            use_tc_tiling_on_sc=False, needs_layout_passes=False),
    )
    def gather2(idx_hbm, p0_hbm, p1_hbm, out_hbm, idx_v, out_v, sem):
        wid = lax.axis_index("s") * NC + lax.axis_index("c")
        base = wid * bw
        for f in range(fields):
            pltpu.sync_copy(idx_hbm.at[f, pl.ds(base, bw)], idx_v.at[f])

        def fire(j, _):
            f = j // qn
            q = j % qn
            ids = idx_v.at[f, pl.ds(q * G, G)]
            pltpu.async_copy(p0_hbm.at[ids], out_v.at[f, 0, pl.ds(q * G, G)],
                             sem)
            pltpu.async_copy(p1_hbm.at[ids], out_v.at[f, 1, pl.ds(q * G, G)],
                             sem)
            return 0

        lax.fori_loop(0, ch, fire, 0)

        def drain(j, _):
            pltpu.make_async_copy(
                p0_hbm.at[idx_v.at[0, pl.ds(0, G)]],
                out_v.at[0, 0, pl.ds(0, G)], sem).wait()
            return 0

        lax.fori_loop(0, 2 * ch, drain, 0)

        for f in range(fields):
            for o in range(OUT_DIM):
                pltpu.sync_copy(out_v.at[f, o],
                                out_hbm.at[f, o, pl.ds(base, bw)])

    return gather2


def kernel(data, table, W, b):
    batch, fields = data.shape
    table_t = table.T
    data_t = data.astype(jnp.int32).T
    p0, p1 = _make_project(table.shape[0])(
        table_t, W.astype(jnp.bfloat16), b.reshape(OUT_DIM, 1))
    out3 = _make_gather(fields, batch)(data_t, p0, p1)
    return out3.transpose(2, 0, 1)
